# baseline (speedup 1.0000x reference)
import functools

import jax
import jax.numpy as jnp
from jax import lax
from jax.experimental import pallas as pl
from jax.experimental.pallas import tpu as pltpu, tpu_sc as plsc

H = 1024
OUT = 1024
E = 16
T = 8192
BLK = 256
NB = 80
P = NB * BLK
NW = 32
TPW = T // NW
CH = 16
NCH = TPW // CH


@functools.lru_cache(maxsize=None)
def _mesh():
    return plsc.VectorSubcoreMesh(
        core_axis_name="c", subcore_axis_name="s", num_cores=2,
        num_subcores=16)


def _wid():
    return lax.axis_index("c") * 16 + lax.axis_index("s")


def _router_body(x_ref, gw_ref, gb_ref, logits_ref):
    logits_ref[...] = lax.dot_general(
        x_ref[...], gw_ref[...], (((1,), (1,)), ((), ())),
        preferred_element_type=jnp.float32) + gb_ref[...]


def _router(hs, gate_w, gate_b):
    return pl.pallas_call(
        _router_body,
        grid=(8,),
        in_specs=[
            pl.BlockSpec((T // 8, H), lambda i: (i, 0)),
            pl.BlockSpec((E, H), lambda i: (0, 0)),
            pl.BlockSpec((1, E), lambda i: (0, 0)),
        ],
        out_specs=pl.BlockSpec((T // 8, E), lambda i: (i, 0)),
        out_shape=jax.ShapeDtypeStruct((T, E), jnp.float32),
    )(hs, gate_w, gate_b.reshape(1, E))


def _k2a_body(logits_hbm, e0_hbm, e1_hbm, w0_hbm, w1_hbm, hist_hbm,
              lg_v, e0_v, e1_v, w0_v, w1_v, hist_v):
    wid = _wid()
    base = wid * TPW
    pltpu.sync_copy(logits_hbm.at[pl.ds(base, TPW)], lg_v)
    lanes = lax.iota(jnp.int32, 16)

    neg = jnp.float32(-jnp.inf)

    def chunk(c, hist):
        a0 = jnp.zeros((16,), jnp.int32)
        a1 = jnp.zeros((16,), jnp.int32)
        w0 = jnp.zeros((16,), jnp.float32)
        w1 = jnp.zeros((16,), jnp.float32)
        for r in range(CH):
            l = lg_v[c * CH + r]
            m0 = jnp.max(l)
            a0s = jnp.min(jnp.where(l == m0, lanes, E))
            lm = jnp.where(lanes == a0s, neg, l)
            m1 = jnp.max(lm)
            a1s = jnp.min(jnp.where(lm == m1, lanes, E))
            t = jnp.exp(jnp.broadcast_to(m1 - m0, (16,)))
            s = 1.0 / (1.0 + t)
            a0 = jnp.where(lanes == r, a0s, a0)
            a1 = jnp.where(lanes == r, a1s, a1)
            w0 = jnp.where(lanes == r, s, w0)
            w1 = jnp.where(lanes == r, t * s, w1)
        e0_v[pl.ds(c * CH, CH)] = a0
        e1_v[pl.ds(c * CH, CH)] = a1
        w0_v[pl.ds(c * CH, CH)] = w0
        w1_v[pl.ds(c * CH, CH)] = w1
        for e in range(E):
            n = jnp.sum((a0 == e).astype(jnp.int32)) + jnp.sum(
                (a1 == e).astype(jnp.int32))
            hist = hist + jnp.where(lanes == e, n, 0)
        return hist

    hist = lax.fori_loop(0, NCH, chunk, jnp.zeros((16,), jnp.int32))
    hist_v[...] = hist
    pltpu.sync_copy(e0_v, e0_hbm.at[pl.ds(base, TPW)])
    pltpu.sync_copy(e1_v, e1_hbm.at[pl.ds(base, TPW)])
    pltpu.sync_copy(w0_v, w0_hbm.at[pl.ds(base, TPW)])
    pltpu.sync_copy(w1_v, w1_hbm.at[pl.ds(base, TPW)])
    pltpu.sync_copy(hist_v, hist_hbm.at[pl.ds(wid * E, E)])


@functools.lru_cache(maxsize=None)
def _get_k2a():
    return pl.kernel(
        _k2a_body,
        out_type=[
            jax.ShapeDtypeStruct((T,), jnp.int32),
            jax.ShapeDtypeStruct((T,), jnp.int32),
            jax.ShapeDtypeStruct((T,), jnp.float32),
            jax.ShapeDtypeStruct((T,), jnp.float32),
            jax.ShapeDtypeStruct((NW * E,), jnp.int32),
        ],
        mesh=_mesh(),
        compiler_params=pltpu.CompilerParams(needs_layout_passes=False),
        scratch_types=[
            pltpu.VMEM((TPW, E), jnp.float32),
            pltpu.VMEM((TPW,), jnp.int32),
            pltpu.VMEM((TPW,), jnp.int32),
            pltpu.VMEM((TPW,), jnp.float32),
            pltpu.VMEM((TPW,), jnp.float32),
            pltpu.VMEM((E,), jnp.int32),
        ],
    )


def _k2b_body(x_hbm, e0_hbm, e1_hbm, hist_hbm, dst0_hbm, dst1_hbm, blke_hbm,
              blkv_hbm, xs_hbm, hist_v, e0_v, e1_v, d0_v, d1_v, blke_v,
              blkv_v, rows_v, sem, sem2):
    wid = _wid()
    base = wid * TPW
    pltpu.sync_copy(hist_hbm, hist_v)
    pltpu.sync_copy(e0_hbm.at[pl.ds(base, TPW)], e0_v)
    pltpu.sync_copy(e1_hbm.at[pl.ds(base, TPW)], e1_v)
    lanes = lax.iota(jnp.int32, 16)

    tot = jnp.zeros((16,), jnp.int32)
    mybase = jnp.zeros((16,), jnp.int32)
    for wj in range(NW):
        row = hist_v[pl.ds(wj * E, E)]
        tot = tot + row
        mybase = mybase + jnp.where(jnp.int32(wj) < wid, row, 0)
    padded = ((tot + (BLK - 1)) >> 8) << 8
    cum = plsc.cumsum(padded)
    start = cum - padded
    basepos = start + mybase

    def chunk(c, cnt):
        k0 = e0_v[pl.ds(c * CH, CH)]
        k1 = e1_v[pl.ds(c * CH, CH)]
        d0 = jnp.zeros((16,), jnp.int32)
        d1 = jnp.zeros((16,), jnp.int32)
        for e in range(E):
            cnt_e = jnp.sum(jnp.where(lanes == e, cnt, 0))
            m0i = (k0 == e).astype(jnp.int32)
            ex0 = plsc.cumsum(m0i) - m0i
            d0 = d0 + (cnt_e + ex0) * m0i
            n0 = jnp.sum(m0i)
            m1i = (k1 == e).astype(jnp.int32)
            ex1 = plsc.cumsum(m1i) - m1i
            d1 = d1 + (cnt_e + n0 + ex1) * m1i
            n1 = jnp.sum(m1i)
            cnt = cnt + jnp.where(lanes == e, n0 + n1, 0)
        d0_v[pl.ds(c * CH, CH)] = d0
        d1_v[pl.ds(c * CH, CH)] = d1
        return cnt

    lax.fori_loop(0, NCH, chunk, basepos)
    pltpu.sync_copy(d0_v, dst0_hbm.at[pl.ds(base, TPW)])
    pltpu.sync_copy(d1_v, dst1_hbm.at[pl.ds(base, TPW)])

    def dchunk(c, _):
        pltpu.sync_copy(x_hbm.at[pl.ds(base + c * CH, CH)], rows_v)
        i0 = d0_v[pl.ds(c * CH, CH)]
        i1 = d1_v[pl.ds(c * CH, CH)]
        cp0 = pltpu.make_async_copy(rows_v, xs_hbm.at[i0], sem)
        cp1 = pltpu.make_async_copy(rows_v, xs_hbm.at[i1], sem2)
        cp0.start()
        cp1.start()
        cp0.wait()
        cp1.wait()
        return 0

    lax.fori_loop(0, NCH, dchunk, 0)

    @pl.when(wid == 0)
    def _blocks():
        total_pad = jnp.sum(padded)
        for g in range(NB // 16):
            pos = (lax.iota(jnp.int32, 16) + g * 16) * BLK
            be = jnp.zeros((16,), jnp.int32)
            for e in range(E):
                end_e = jnp.sum(jnp.where(lanes == e, cum, 0))
                be = be + (pos >= end_e).astype(jnp.int32)
            blke_v[pl.ds(g * 16, 16)] = jnp.minimum(be, E - 1)
            blkv_v[pl.ds(g * 16, 16)] = (pos < total_pad).astype(jnp.int32)
        pltpu.sync_copy(blke_v, blke_hbm)
        pltpu.sync_copy(blkv_v, blkv_hbm)


@functools.lru_cache(maxsize=None)
def _get_k2b():
    return pl.kernel(
        _k2b_body,
        out_type=[
            jax.ShapeDtypeStruct((T,), jnp.int32),
            jax.ShapeDtypeStruct((T,), jnp.int32),
            jax.ShapeDtypeStruct((NB,), jnp.int32),
            jax.ShapeDtypeStruct((NB,), jnp.int32),
            jax.ShapeDtypeStruct((P, H // 2), jnp.int32),
        ],
        mesh=_mesh(),
        compiler_params=pltpu.CompilerParams(needs_layout_passes=False),
        scratch_types=[
            pltpu.VMEM((NW * E,), jnp.int32),
            pltpu.VMEM((TPW,), jnp.int32),
            pltpu.VMEM((TPW,), jnp.int32),
            pltpu.VMEM((TPW,), jnp.int32),
            pltpu.VMEM((TPW,), jnp.int32),
            pltpu.VMEM((NB,), jnp.int32),
            pltpu.VMEM((NB,), jnp.int32),
            pltpu.VMEM((CH, H // 2), jnp.int32),
            pltpu.SemaphoreType.DMA,
            pltpu.SemaphoreType.DMA,
        ],
    )


def _gmm_body(be_ref, bv_ref, xs_ref, ew_ref, eb_ref, y_ref):
    i = pl.program_id(0)

    @pl.when(bv_ref[i] != 0)
    def _():
        y_ref[...] = lax.dot_general(
            xs_ref[...], ew_ref[0], (((1,), (1,)), ((), ())),
            preferred_element_type=jnp.float32) + eb_ref[0]


def _gmm(blke, blkv, xs, expert_w, expert_b):
    grid_spec = pltpu.PrefetchScalarGridSpec(
        num_scalar_prefetch=2,
        grid=(NB,),
        in_specs=[
            pl.BlockSpec((BLK, H), lambda i, be, bv: (i, 0)),
            pl.BlockSpec((1, OUT, H), lambda i, be, bv: (be[i], 0, 0)),
            pl.BlockSpec((1, 1, OUT), lambda i, be, bv: (be[i], 0, 0)),
        ],
        out_specs=pl.BlockSpec((BLK, OUT), lambda i, be, bv: (i, 0)),
    )
    return pl.pallas_call(
        _gmm_body,
        grid_spec=grid_spec,
        out_shape=jax.ShapeDtypeStruct((P, OUT), jnp.float32),
        compiler_params=pltpu.CompilerParams(
            dimension_semantics=("arbitrary",)),
    )(blke, blkv, xs, expert_w.astype(jnp.bfloat16),
      expert_b.reshape(E, 1, OUT))


def _k5_body(y_hbm, dst0_hbm, dst1_hbm, w0_hbm, w1_hbm, out_hbm,
             d0_v, d1_v, w0_v, w1_v, y0a_v, y1a_v, y0b_v, y1b_v, o_v,
             sem_a, sem_b):
    wid = _wid()
    base = wid * TPW
    pltpu.sync_copy(dst0_hbm.at[pl.ds(base, TPW)], d0_v)
    pltpu.sync_copy(dst1_hbm.at[pl.ds(base, TPW)], d1_v)
    pltpu.sync_copy(w0_hbm.at[pl.ds(base, TPW)], w0_v)
    pltpu.sync_copy(w1_hbm.at[pl.ds(base, TPW)], w1_v)
    lanes = lax.iota(jnp.int32, 16)

    UNROLL = 8

    def issue(c, yb0, yb1, sem):
        i0 = d0_v[pl.ds(c * CH, CH)]
        i1 = d1_v[pl.ds(c * CH, CH)]
        pltpu.make_async_copy(y_hbm.at[i0], yb0, sem).start()
        pltpu.make_async_copy(y_hbm.at[i1], yb1, sem).start()

    def wait_pair(c, yb0, yb1, sem):
        i0 = d0_v[pl.ds(c * CH, CH)]
        pltpu.make_async_copy(y_hbm.at[i0], yb0, sem).wait()
        pltpu.make_async_copy(y_hbm.at[i0], yb1, sem).wait()

    def compute(c, yb0, yb1):
        wa = w0_v[pl.ds(c * CH, CH)]
        wb = w1_v[pl.ds(c * CH, CH)]
        for r in range(CH):
            w0r = jnp.sum(jnp.where(lanes == r, wa, 0.0))
            w1r = jnp.sum(jnp.where(lanes == r, wb, 0.0))

            def hloop(h, _, _r=r, _w0=w0r, _w1=w1r):
                for u in range(UNROLL):
                    sl = pl.ds((h * UNROLL + u) * 16, 16)
                    o_v[_r, sl] = yb0[_r, sl] * _w0 + yb1[_r, sl] * _w1
                return 0

            lax.fori_loop(0, OUT // (16 * UNROLL), hloop, 0)
        pltpu.sync_copy(o_v, out_hbm.at[pl.ds(base + c * CH, CH)])

    issue(0, y0a_v, y1a_v, sem_a)

    def chunk(c, _):
        @pl.when(c % 2 == 0)
        def _even():
            wait_pair(c, y0a_v, y1a_v, sem_a)

            @pl.when(c + 1 < NCH)
            def _():
                issue(c + 1, y0b_v, y1b_v, sem_b)

            compute(c, y0a_v, y1a_v)

        @pl.when(c % 2 != 0)
        def _odd():
            wait_pair(c, y0b_v, y1b_v, sem_b)

            @pl.when(c + 1 < NCH)
            def _():
                issue(c + 1, y0a_v, y1a_v, sem_a)

            compute(c, y0b_v, y1b_v)

        return 0

    lax.fori_loop(0, NCH, chunk, 0)


@functools.lru_cache(maxsize=None)
def _get_k5():
    return pl.kernel(
        _k5_body,
        out_type=jax.ShapeDtypeStruct((T, OUT), jnp.float32),
        mesh=_mesh(),
        compiler_params=pltpu.CompilerParams(needs_layout_passes=False),
        scratch_types=[
            pltpu.VMEM((TPW,), jnp.int32),
            pltpu.VMEM((TPW,), jnp.int32),
            pltpu.VMEM((TPW,), jnp.float32),
            pltpu.VMEM((TPW,), jnp.float32),
            pltpu.VMEM((CH, OUT), jnp.float32),
            pltpu.VMEM((CH, OUT), jnp.float32),
            pltpu.VMEM((CH, OUT), jnp.float32),
            pltpu.VMEM((CH, OUT), jnp.float32),
            pltpu.VMEM((CH, OUT), jnp.float32),
            pltpu.SemaphoreType.DMA,
            pltpu.SemaphoreType.DMA,
        ],
    )


def kernel(x, gate_w, gate_b, expert_w, expert_b):
    B, S, Hd = x.shape
    hs = x.reshape(-1, Hd)
    logits = _router(hs, gate_w, gate_b)
    e0, e1, w0, w1, hist = _get_k2a()(logits)
    hs_words = lax.bitcast_convert_type(
        hs.astype(jnp.bfloat16).reshape(T, H // 2, 2), jnp.int32)
    dst0, dst1, blke, blkv, xs_words = _get_k2b()(hs_words, e0, e1, hist)
    xs = lax.bitcast_convert_type(
        xs_words, jnp.bfloat16).reshape(P, H)
    y = _gmm(blke, blkv, xs, expert_w, expert_b)
    out = _get_k5()(y, dst0, dst1, w0, w1)
    return out.reshape(B, S, OUT), logits


# speedup vs baseline: 3.0780x; 3.0780x over previous
import functools

import jax
import jax.numpy as jnp
from jax import lax
from jax.experimental import pallas as pl
from jax.experimental.pallas import tpu as pltpu, tpu_sc as plsc

H = 1024
OUT = 1024
E = 16
T = 8192
BLK = 256
NB = 80
P = NB * BLK
NW = 32
TPW = T // NW
CH = 16
NCH = TPW // CH


@functools.lru_cache(maxsize=None)
def _mesh():
    return plsc.VectorSubcoreMesh(
        core_axis_name="c", subcore_axis_name="s", num_cores=2,
        num_subcores=16)


def _wid():
    return lax.axis_index("c") * 16 + lax.axis_index("s")


def _router_body(x_ref, gw_ref, gb_ref, logits_ref):
    logits_ref[...] = lax.dot_general(
        x_ref[...], gw_ref[...], (((1,), (1,)), ((), ())),
        preferred_element_type=jnp.float32) + gb_ref[...]


def _router(hs, gate_w, gate_b):
    return pl.pallas_call(
        _router_body,
        grid=(8,),
        in_specs=[
            pl.BlockSpec((T // 8, H), lambda i: (i, 0)),
            pl.BlockSpec((E, H), lambda i: (0, 0)),
            pl.BlockSpec((1, E), lambda i: (0, 0)),
        ],
        out_specs=pl.BlockSpec((T // 8, E), lambda i: (i, 0)),
        out_shape=jax.ShapeDtypeStruct((T, E), jnp.float32),
    )(hs, gate_w, gate_b.reshape(1, E))


def _k2a_body(logits_hbm, e0_hbm, e1_hbm, w0_hbm, w1_hbm, hist_hbm,
              lg_v, e0_v, e1_v, w0_v, w1_v, hist_v):
    wid = _wid()
    base = wid * TPW
    pltpu.sync_copy(logits_hbm.at[pl.ds(base * E, TPW * E)], lg_v)
    lanes = lax.iota(jnp.int32, 16)

    neg = jnp.float32(-jnp.inf)

    def chunk(c, hist):
        a0 = jnp.zeros((16,), jnp.int32)
        a1 = jnp.zeros((16,), jnp.int32)
        w0 = jnp.zeros((16,), jnp.float32)
        w1 = jnp.zeros((16,), jnp.float32)
        for r in range(CH):
            l = lg_v[pl.ds((c * CH + r) * E, E)]
            m0 = jnp.max(l)
            a0s = jnp.min(jnp.where(l == m0, lanes, E))
            lm = jnp.where(lanes == a0s, neg, l)
            m1 = jnp.max(lm)
            a1s = jnp.min(jnp.where(lm == m1, lanes, E))
            t = jnp.exp(jnp.broadcast_to(m1 - m0, (16,)))
            s = 1.0 / (1.0 + t)
            a0 = jnp.where(lanes == r, a0s, a0)
            a1 = jnp.where(lanes == r, a1s, a1)
            w0 = jnp.where(lanes == r, s, w0)
            w1 = jnp.where(lanes == r, t * s, w1)
        e0_v[pl.ds(c * CH, CH)] = a0
        e1_v[pl.ds(c * CH, CH)] = a1
        w0_v[pl.ds(c * CH, CH)] = w0
        w1_v[pl.ds(c * CH, CH)] = w1
        for e in range(E):
            n = jnp.sum((a0 == e).astype(jnp.int32)) + jnp.sum(
                (a1 == e).astype(jnp.int32))
            hist = hist + jnp.where(lanes == e, n, 0)
        return hist

    hist = lax.fori_loop(0, NCH, chunk, jnp.zeros((16,), jnp.int32))
    hist_v[...] = hist
    pltpu.sync_copy(e0_v, e0_hbm.at[pl.ds(base, TPW)])
    pltpu.sync_copy(e1_v, e1_hbm.at[pl.ds(base, TPW)])
    pltpu.sync_copy(w0_v, w0_hbm.at[pl.ds(base, TPW)])
    pltpu.sync_copy(w1_v, w1_hbm.at[pl.ds(base, TPW)])
    pltpu.sync_copy(hist_v, hist_hbm.at[pl.ds(wid * E, E)])


@functools.lru_cache(maxsize=None)
def _get_k2a():
    return pl.kernel(
        _k2a_body,
        out_type=[
            jax.ShapeDtypeStruct((T,), jnp.int32),
            jax.ShapeDtypeStruct((T,), jnp.int32),
            jax.ShapeDtypeStruct((T,), jnp.float32),
            jax.ShapeDtypeStruct((T,), jnp.float32),
            jax.ShapeDtypeStruct((NW * E,), jnp.int32),
        ],
        mesh=_mesh(),
        compiler_params=pltpu.CompilerParams(needs_layout_passes=False),
        scratch_types=[
            pltpu.VMEM((TPW * E,), jnp.float32),
            pltpu.VMEM((TPW,), jnp.int32),
            pltpu.VMEM((TPW,), jnp.int32),
            pltpu.VMEM((TPW,), jnp.float32),
            pltpu.VMEM((TPW,), jnp.float32),
            pltpu.VMEM((E,), jnp.int32),
        ],
    )


def _k2b_body(x_hbm, e0_hbm, e1_hbm, hist_hbm, dst0_hbm, dst1_hbm, blke_hbm,
              blkv_hbm, xs_hbm, hist_v, e0_v, e1_v, d0_v, d1_v, blke_v,
              blkv_v, rows_v, sem, sem2):
    wid = _wid()
    base = wid * TPW
    pltpu.sync_copy(hist_hbm, hist_v)
    pltpu.sync_copy(e0_hbm.at[pl.ds(base, TPW)], e0_v)
    pltpu.sync_copy(e1_hbm.at[pl.ds(base, TPW)], e1_v)
    lanes = lax.iota(jnp.int32, 16)

    tot = jnp.zeros((16,), jnp.int32)
    mybase = jnp.zeros((16,), jnp.int32)
    for wj in range(NW):
        row = hist_v[pl.ds(wj * E, E)]
        tot = tot + row
        mybase = mybase + jnp.where(jnp.int32(wj) < wid, row, 0)
    padded = ((tot + (BLK - 1)) >> 8) << 8
    cum = plsc.cumsum(padded)
    start = cum - padded
    basepos = start + mybase

    def chunk(c, cnt):
        k0 = e0_v[pl.ds(c * CH, CH)]
        k1 = e1_v[pl.ds(c * CH, CH)]
        d0 = jnp.zeros((16,), jnp.int32)
        d1 = jnp.zeros((16,), jnp.int32)
        for e in range(E):
            cnt_e = jnp.sum(jnp.where(lanes == e, cnt, 0))
            m0i = (k0 == e).astype(jnp.int32)
            ex0 = plsc.cumsum(m0i) - m0i
            d0 = d0 + (cnt_e + ex0) * m0i
            n0 = jnp.sum(m0i)
            m1i = (k1 == e).astype(jnp.int32)
            ex1 = plsc.cumsum(m1i) - m1i
            d1 = d1 + (cnt_e + n0 + ex1) * m1i
            n1 = jnp.sum(m1i)
            cnt = cnt + jnp.where(lanes == e, n0 + n1, 0)
        d0_v[pl.ds(c * CH, CH)] = d0
        d1_v[pl.ds(c * CH, CH)] = d1
        return cnt

    lax.fori_loop(0, NCH, chunk, basepos)
    pltpu.sync_copy(d0_v, dst0_hbm.at[pl.ds(base, TPW)])
    pltpu.sync_copy(d1_v, dst1_hbm.at[pl.ds(base, TPW)])

    def dchunk(c, _):
        pltpu.sync_copy(x_hbm.at[pl.ds(base + c * CH, CH)], rows_v)
        i0 = d0_v[pl.ds(c * CH, CH)]
        i1 = d1_v[pl.ds(c * CH, CH)]
        cp0 = pltpu.make_async_copy(rows_v, xs_hbm.at[i0], sem)
        cp1 = pltpu.make_async_copy(rows_v, xs_hbm.at[i1], sem2)
        cp0.start()
        cp1.start()
        cp0.wait()
        cp1.wait()
        return 0

    lax.fori_loop(0, NCH, dchunk, 0)

    @pl.when(wid == 0)
    def _blocks():
        total_pad = jnp.sum(padded)
        for g in range(NB // 16):
            pos = (lax.iota(jnp.int32, 16) + g * 16) * BLK
            be = jnp.zeros((16,), jnp.int32)
            for e in range(E):
                end_e = jnp.sum(jnp.where(lanes == e, cum, 0))
                be = be + (pos >= end_e).astype(jnp.int32)
            blke_v[pl.ds(g * 16, 16)] = jnp.minimum(be, E - 1)
            blkv_v[pl.ds(g * 16, 16)] = (pos < total_pad).astype(jnp.int32)
        pltpu.sync_copy(blke_v, blke_hbm)
        pltpu.sync_copy(blkv_v, blkv_hbm)


@functools.lru_cache(maxsize=None)
def _get_k2b():
    return pl.kernel(
        _k2b_body,
        out_type=[
            jax.ShapeDtypeStruct((T,), jnp.int32),
            jax.ShapeDtypeStruct((T,), jnp.int32),
            jax.ShapeDtypeStruct((NB,), jnp.int32),
            jax.ShapeDtypeStruct((NB,), jnp.int32),
            jax.ShapeDtypeStruct((P, H), jnp.float32),
        ],
        mesh=_mesh(),
        compiler_params=pltpu.CompilerParams(needs_layout_passes=False),
        scratch_types=[
            pltpu.VMEM((NW * E,), jnp.int32),
            pltpu.VMEM((TPW,), jnp.int32),
            pltpu.VMEM((TPW,), jnp.int32),
            pltpu.VMEM((TPW,), jnp.int32),
            pltpu.VMEM((TPW,), jnp.int32),
            pltpu.VMEM((NB,), jnp.int32),
            pltpu.VMEM((NB,), jnp.int32),
            pltpu.VMEM((CH, H), jnp.float32),
            pltpu.SemaphoreType.DMA,
            pltpu.SemaphoreType.DMA,
        ],
    )


def _gmm_body(be_ref, bv_ref, xs_ref, ew_ref, eb_ref, y_ref):
    i = pl.program_id(0)

    @pl.when(bv_ref[i] != 0)
    def _():
        y_ref[...] = lax.dot_general(
            xs_ref[...], ew_ref[0], (((1,), (1,)), ((), ())),
            preferred_element_type=jnp.float32) + eb_ref[0]


def _gmm(blke, blkv, xs, expert_w, expert_b):
    grid_spec = pltpu.PrefetchScalarGridSpec(
        num_scalar_prefetch=2,
        grid=(NB,),
        in_specs=[
            pl.BlockSpec((BLK, H), lambda i, be, bv: (i, 0)),
            pl.BlockSpec((1, OUT, H), lambda i, be, bv: (be[i], 0, 0)),
            pl.BlockSpec((1, 1, OUT), lambda i, be, bv: (be[i], 0, 0)),
        ],
        out_specs=pl.BlockSpec((BLK, OUT), lambda i, be, bv: (i, 0)),
    )
    return pl.pallas_call(
        _gmm_body,
        grid_spec=grid_spec,
        out_shape=jax.ShapeDtypeStruct((P, OUT), jnp.float32),
        compiler_params=pltpu.CompilerParams(
            dimension_semantics=("arbitrary",)),
    )(blke, blkv, xs, expert_w, expert_b.reshape(E, 1, OUT))


def _k5_body(y_hbm, dst0_hbm, dst1_hbm, w0_hbm, w1_hbm, out_hbm,
             d0_v, d1_v, w0_v, w1_v, y0a_v, y1a_v, y0b_v, y1b_v, o_v,
             sem_a, sem_b):
    wid = _wid()
    base = wid * TPW
    pltpu.sync_copy(dst0_hbm.at[pl.ds(base, TPW)], d0_v)
    pltpu.sync_copy(dst1_hbm.at[pl.ds(base, TPW)], d1_v)
    pltpu.sync_copy(w0_hbm.at[pl.ds(base, TPW)], w0_v)
    pltpu.sync_copy(w1_hbm.at[pl.ds(base, TPW)], w1_v)
    lanes = lax.iota(jnp.int32, 16)

    UNROLL = 8

    def issue(c, yb0, yb1, sem):
        i0 = d0_v[pl.ds(c * CH, CH)]
        i1 = d1_v[pl.ds(c * CH, CH)]
        pltpu.make_async_copy(y_hbm.at[i0], yb0, sem).start()
        pltpu.make_async_copy(y_hbm.at[i1], yb1, sem).start()

    def wait_pair(c, yb0, yb1, sem):
        i0 = d0_v[pl.ds(c * CH, CH)]
        pltpu.make_async_copy(y_hbm.at[i0], yb0, sem).wait()
        pltpu.make_async_copy(y_hbm.at[i0], yb1, sem).wait()

    def compute(c, yb0, yb1):
        wa = w0_v[pl.ds(c * CH, CH)]
        wb = w1_v[pl.ds(c * CH, CH)]
        for r in range(CH):
            w0r = jnp.sum(jnp.where(lanes == r, wa, 0.0))
            w1r = jnp.sum(jnp.where(lanes == r, wb, 0.0))

            def hloop(h, _, _r=r, _w0=w0r, _w1=w1r):
                for u in range(UNROLL):
                    sl = pl.ds((h * UNROLL + u) * 16, 16)
                    o_v[_r, sl] = yb0[_r, sl] * _w0 + yb1[_r, sl] * _w1
                return 0

            lax.fori_loop(0, OUT // (16 * UNROLL), hloop, 0)
        pltpu.sync_copy(o_v, out_hbm.at[pl.ds(base + c * CH, CH)])

    issue(0, y0a_v, y1a_v, sem_a)

    def chunk(c, _):
        @pl.when(c % 2 == 0)
        def _even():
            wait_pair(c, y0a_v, y1a_v, sem_a)

            @pl.when(c + 1 < NCH)
            def _():
                issue(c + 1, y0b_v, y1b_v, sem_b)

            compute(c, y0a_v, y1a_v)

        @pl.when(c % 2 != 0)
        def _odd():
            wait_pair(c, y0b_v, y1b_v, sem_b)

            @pl.when(c + 1 < NCH)
            def _():
                issue(c + 1, y0a_v, y1a_v, sem_a)

            compute(c, y0b_v, y1b_v)

        return 0

    lax.fori_loop(0, NCH, chunk, 0)


@functools.lru_cache(maxsize=None)
def _get_k5():
    return pl.kernel(
        _k5_body,
        out_type=jax.ShapeDtypeStruct((T, OUT), jnp.float32),
        mesh=_mesh(),
        compiler_params=pltpu.CompilerParams(needs_layout_passes=False),
        scratch_types=[
            pltpu.VMEM((TPW,), jnp.int32),
            pltpu.VMEM((TPW,), jnp.int32),
            pltpu.VMEM((TPW,), jnp.float32),
            pltpu.VMEM((TPW,), jnp.float32),
            pltpu.VMEM((CH, OUT), jnp.float32),
            pltpu.VMEM((CH, OUT), jnp.float32),
            pltpu.VMEM((CH, OUT), jnp.float32),
            pltpu.VMEM((CH, OUT), jnp.float32),
            pltpu.VMEM((CH, OUT), jnp.float32),
            pltpu.SemaphoreType.DMA,
            pltpu.SemaphoreType.DMA,
        ],
    )


def kernel(x, gate_w, gate_b, expert_w, expert_b):
    B, S, Hd = x.shape
    hs = x.reshape(-1, Hd)
    logits = _router(hs, gate_w, gate_b)
    e0, e1, w0, w1, hist = _get_k2a()(logits.reshape(-1))
    dst0, dst1, blke, blkv, xs = _get_k2b()(hs, e0, e1, hist)
    y = _gmm(blke, blkv, xs, expert_w, expert_b)
    out = _get_k5()(y, dst0, dst1, w0, w1)
    return out.reshape(B, S, OUT), logits


# speedup vs baseline: 3.2326x; 1.0502x over previous
import functools

import jax
import jax.numpy as jnp
from jax import lax
from jax.experimental import pallas as pl
from jax.experimental.pallas import tpu as pltpu, tpu_sc as plsc

H = 1024
OUT = 1024
E = 16
T = 8192
BLK = 512
BLK_SHIFT = 9
NB = 48
P = NB * BLK
NW = 32
TPW = T // NW
CH = 16
NCH = TPW // CH


@functools.lru_cache(maxsize=None)
def _mesh():
    return plsc.VectorSubcoreMesh(
        core_axis_name="c", subcore_axis_name="s", num_cores=2,
        num_subcores=16)


def _wid():
    return lax.axis_index("c") * 16 + lax.axis_index("s")


def _router_body(x_ref, gw_ref, gb_ref, logits_ref):
    logits_ref[...] = lax.dot_general(
        x_ref[...], gw_ref[...], (((1,), (1,)), ((), ())),
        preferred_element_type=jnp.float32) + gb_ref[...]


def _router(hs, gate_w, gate_b):
    return pl.pallas_call(
        _router_body,
        grid=(8,),
        in_specs=[
            pl.BlockSpec((T // 8, H), lambda i: (i, 0)),
            pl.BlockSpec((E, H), lambda i: (0, 0)),
            pl.BlockSpec((1, E), lambda i: (0, 0)),
        ],
        out_specs=pl.BlockSpec((T // 8, E), lambda i: (i, 0)),
        out_shape=jax.ShapeDtypeStruct((T, E), jnp.float32),
    )(hs, gate_w, gate_b.reshape(1, E))


def _k2a_body(logits_hbm, e0_hbm, e1_hbm, w0_hbm, w1_hbm, hist_hbm,
              lg_v, e0_v, e1_v, w0_v, w1_v, hist_v):
    wid = _wid()
    base = wid * TPW
    pltpu.sync_copy(logits_hbm.at[pl.ds(base * E, TPW * E)], lg_v)
    lanes = lax.iota(jnp.int32, 16)

    neg = jnp.float32(-jnp.inf)

    def chunk(c, hist):
        a0 = jnp.zeros((16,), jnp.int32)
        a1 = jnp.zeros((16,), jnp.int32)
        w0 = jnp.zeros((16,), jnp.float32)
        w1 = jnp.zeros((16,), jnp.float32)
        for r in range(CH):
            l = lg_v[pl.ds((c * CH + r) * E, E)]
            m0 = jnp.max(l)
            a0s = jnp.min(jnp.where(l == m0, lanes, E))
            lm = jnp.where(lanes == a0s, neg, l)
            m1 = jnp.max(lm)
            a1s = jnp.min(jnp.where(lm == m1, lanes, E))
            t = jnp.exp(jnp.broadcast_to(m1 - m0, (16,)))
            s = 1.0 / (1.0 + t)
            a0 = jnp.where(lanes == r, a0s, a0)
            a1 = jnp.where(lanes == r, a1s, a1)
            w0 = jnp.where(lanes == r, s, w0)
            w1 = jnp.where(lanes == r, t * s, w1)
        e0_v[pl.ds(c * CH, CH)] = a0
        e1_v[pl.ds(c * CH, CH)] = a1
        w0_v[pl.ds(c * CH, CH)] = w0
        w1_v[pl.ds(c * CH, CH)] = w1
        for e in range(E):
            n = jnp.sum((a0 == e).astype(jnp.int32)) + jnp.sum(
                (a1 == e).astype(jnp.int32))
            hist = hist + jnp.where(lanes == e, n, 0)
        return hist

    hist = lax.fori_loop(0, NCH, chunk, jnp.zeros((16,), jnp.int32))
    hist_v[...] = hist
    pltpu.sync_copy(e0_v, e0_hbm.at[pl.ds(base, TPW)])
    pltpu.sync_copy(e1_v, e1_hbm.at[pl.ds(base, TPW)])
    pltpu.sync_copy(w0_v, w0_hbm.at[pl.ds(base, TPW)])
    pltpu.sync_copy(w1_v, w1_hbm.at[pl.ds(base, TPW)])
    pltpu.sync_copy(hist_v, hist_hbm.at[pl.ds(wid * E, E)])


@functools.lru_cache(maxsize=None)
def _get_k2a():
    return pl.kernel(
        _k2a_body,
        out_type=[
            jax.ShapeDtypeStruct((T,), jnp.int32),
            jax.ShapeDtypeStruct((T,), jnp.int32),
            jax.ShapeDtypeStruct((T,), jnp.float32),
            jax.ShapeDtypeStruct((T,), jnp.float32),
            jax.ShapeDtypeStruct((NW * E,), jnp.int32),
        ],
        mesh=_mesh(),
        compiler_params=pltpu.CompilerParams(needs_layout_passes=False),
        scratch_types=[
            pltpu.VMEM((TPW * E,), jnp.float32),
            pltpu.VMEM((TPW,), jnp.int32),
            pltpu.VMEM((TPW,), jnp.int32),
            pltpu.VMEM((TPW,), jnp.float32),
            pltpu.VMEM((TPW,), jnp.float32),
            pltpu.VMEM((E,), jnp.int32),
        ],
    )


def _k2b_body(x_hbm, e0_hbm, e1_hbm, hist_hbm, dst0_hbm, dst1_hbm, blke_hbm,
              blkv_hbm, xs_hbm, hist_v, e0_v, e1_v, d0_v, d1_v, blke_v,
              blkv_v, rows_v, sem, sem2):
    wid = _wid()
    base = wid * TPW
    pltpu.sync_copy(hist_hbm, hist_v)
    pltpu.sync_copy(e0_hbm.at[pl.ds(base, TPW)], e0_v)
    pltpu.sync_copy(e1_hbm.at[pl.ds(base, TPW)], e1_v)
    lanes = lax.iota(jnp.int32, 16)

    tot = jnp.zeros((16,), jnp.int32)
    mybase = jnp.zeros((16,), jnp.int32)
    for wj in range(NW):
        row = hist_v[pl.ds(wj * E, E)]
        tot = tot + row
        mybase = mybase + jnp.where(jnp.int32(wj) < wid, row, 0)
    padded = ((tot + (BLK - 1)) >> BLK_SHIFT) << BLK_SHIFT
    cum = plsc.cumsum(padded)
    start = cum - padded
    basepos = start + mybase

    def chunk(c, cnt):
        k0 = e0_v[pl.ds(c * CH, CH)]
        k1 = e1_v[pl.ds(c * CH, CH)]
        d0 = jnp.zeros((16,), jnp.int32)
        d1 = jnp.zeros((16,), jnp.int32)
        for e in range(E):
            cnt_e = jnp.sum(jnp.where(lanes == e, cnt, 0))
            m0i = (k0 == e).astype(jnp.int32)
            ex0 = plsc.cumsum(m0i) - m0i
            d0 = d0 + (cnt_e + ex0) * m0i
            n0 = jnp.sum(m0i)
            m1i = (k1 == e).astype(jnp.int32)
            ex1 = plsc.cumsum(m1i) - m1i
            d1 = d1 + (cnt_e + n0 + ex1) * m1i
            n1 = jnp.sum(m1i)
            cnt = cnt + jnp.where(lanes == e, n0 + n1, 0)
        d0_v[pl.ds(c * CH, CH)] = d0
        d1_v[pl.ds(c * CH, CH)] = d1
        return cnt

    lax.fori_loop(0, NCH, chunk, basepos)
    pltpu.sync_copy(d0_v, dst0_hbm.at[pl.ds(base, TPW)])
    pltpu.sync_copy(d1_v, dst1_hbm.at[pl.ds(base, TPW)])

    def dchunk(c, _):
        pltpu.sync_copy(x_hbm.at[pl.ds(base + c * CH, CH)], rows_v)
        i0 = d0_v[pl.ds(c * CH, CH)]
        i1 = d1_v[pl.ds(c * CH, CH)]
        cp0 = pltpu.make_async_copy(rows_v, xs_hbm.at[i0], sem)
        cp1 = pltpu.make_async_copy(rows_v, xs_hbm.at[i1], sem2)
        cp0.start()
        cp1.start()
        cp0.wait()
        cp1.wait()
        return 0

    lax.fori_loop(0, NCH, dchunk, 0)

    @pl.when(wid == 0)
    def _blocks():
        total_pad = jnp.sum(padded)
        for g in range(NB // 16):
            pos = (lax.iota(jnp.int32, 16) + g * 16) * BLK
            be = jnp.zeros((16,), jnp.int32)
            for e in range(E):
                end_e = jnp.sum(jnp.where(lanes == e, cum, 0))
                be = be + (pos >= end_e).astype(jnp.int32)
            blke_v[pl.ds(g * 16, 16)] = jnp.minimum(be, E - 1)
            blkv_v[pl.ds(g * 16, 16)] = (pos < total_pad).astype(jnp.int32)
        pltpu.sync_copy(blke_v, blke_hbm)
        pltpu.sync_copy(blkv_v, blkv_hbm)


@functools.lru_cache(maxsize=None)
def _get_k2b():
    return pl.kernel(
        _k2b_body,
        out_type=[
            jax.ShapeDtypeStruct((T,), jnp.int32),
            jax.ShapeDtypeStruct((T,), jnp.int32),
            jax.ShapeDtypeStruct((NB,), jnp.int32),
            jax.ShapeDtypeStruct((NB,), jnp.int32),
            jax.ShapeDtypeStruct((P, H), jnp.float32),
        ],
        mesh=_mesh(),
        compiler_params=pltpu.CompilerParams(needs_layout_passes=False),
        scratch_types=[
            pltpu.VMEM((NW * E,), jnp.int32),
            pltpu.VMEM((TPW,), jnp.int32),
            pltpu.VMEM((TPW,), jnp.int32),
            pltpu.VMEM((TPW,), jnp.int32),
            pltpu.VMEM((TPW,), jnp.int32),
            pltpu.VMEM((NB,), jnp.int32),
            pltpu.VMEM((NB,), jnp.int32),
            pltpu.VMEM((CH, H), jnp.float32),
            pltpu.SemaphoreType.DMA,
            pltpu.SemaphoreType.DMA,
        ],
    )


def _gmm_body(be_ref, bv_ref, xs_ref, ew_ref, eb_ref, y_ref):
    i = pl.program_id(0)

    @pl.when(bv_ref[i] != 0)
    def _():
        y_ref[...] = lax.dot_general(
            xs_ref[...], ew_ref[0], (((1,), (1,)), ((), ())),
            preferred_element_type=jnp.float32) + eb_ref[0]


def _gmm(blke, blkv, xs, expert_w, expert_b):
    grid_spec = pltpu.PrefetchScalarGridSpec(
        num_scalar_prefetch=2,
        grid=(NB,),
        in_specs=[
            pl.BlockSpec((BLK, H), lambda i, be, bv: (i, 0)),
            pl.BlockSpec((1, OUT, H), lambda i, be, bv: (be[i], 0, 0)),
            pl.BlockSpec((1, 1, OUT), lambda i, be, bv: (be[i], 0, 0)),
        ],
        out_specs=pl.BlockSpec((BLK, OUT), lambda i, be, bv: (i, 0)),
    )
    return pl.pallas_call(
        _gmm_body,
        grid_spec=grid_spec,
        out_shape=jax.ShapeDtypeStruct((P, OUT), jnp.float32),
        compiler_params=pltpu.CompilerParams(
            dimension_semantics=("arbitrary",)),
    )(blke, blkv, xs, expert_w, expert_b.reshape(E, 1, OUT))


def _k5_body(y_hbm, dst0_hbm, dst1_hbm, w0_hbm, w1_hbm, out_hbm,
             d0_v, d1_v, w0_v, w1_v, y0a_v, y1a_v, y0b_v, y1b_v, o_v,
             sem_a, sem_b):
    wid = _wid()
    base = wid * TPW
    pltpu.sync_copy(dst0_hbm.at[pl.ds(base, TPW)], d0_v)
    pltpu.sync_copy(dst1_hbm.at[pl.ds(base, TPW)], d1_v)
    pltpu.sync_copy(w0_hbm.at[pl.ds(base, TPW)], w0_v)
    pltpu.sync_copy(w1_hbm.at[pl.ds(base, TPW)], w1_v)
    lanes = lax.iota(jnp.int32, 16)

    UNROLL = 8

    def issue(c, yb0, yb1, sem):
        i0 = d0_v[pl.ds(c * CH, CH)]
        i1 = d1_v[pl.ds(c * CH, CH)]
        pltpu.make_async_copy(y_hbm.at[i0], yb0, sem).start()
        pltpu.make_async_copy(y_hbm.at[i1], yb1, sem).start()

    def wait_pair(c, yb0, yb1, sem):
        i0 = d0_v[pl.ds(c * CH, CH)]
        pltpu.make_async_copy(y_hbm.at[i0], yb0, sem).wait()
        pltpu.make_async_copy(y_hbm.at[i0], yb1, sem).wait()

    def compute(c, yb0, yb1):
        wa = w0_v[pl.ds(c * CH, CH)]
        wb = w1_v[pl.ds(c * CH, CH)]
        for r in range(CH):
            w0r = jnp.sum(jnp.where(lanes == r, wa, 0.0))
            w1r = jnp.sum(jnp.where(lanes == r, wb, 0.0))

            def hloop(h, _, _r=r, _w0=w0r, _w1=w1r):
                for u in range(UNROLL):
                    sl = pl.ds((h * UNROLL + u) * 16, 16)
                    o_v[_r, sl] = yb0[_r, sl] * _w0 + yb1[_r, sl] * _w1
                return 0

            lax.fori_loop(0, OUT // (16 * UNROLL), hloop, 0)
        pltpu.sync_copy(o_v, out_hbm.at[pl.ds(base + c * CH, CH)])

    issue(0, y0a_v, y1a_v, sem_a)

    def chunk(c, _):
        @pl.when(c % 2 == 0)
        def _even():
            wait_pair(c, y0a_v, y1a_v, sem_a)

            @pl.when(c + 1 < NCH)
            def _():
                issue(c + 1, y0b_v, y1b_v, sem_b)

            compute(c, y0a_v, y1a_v)

        @pl.when(c % 2 != 0)
        def _odd():
            wait_pair(c, y0b_v, y1b_v, sem_b)

            @pl.when(c + 1 < NCH)
            def _():
                issue(c + 1, y0a_v, y1a_v, sem_a)

            compute(c, y0b_v, y1b_v)

        return 0

    lax.fori_loop(0, NCH, chunk, 0)


@functools.lru_cache(maxsize=None)
def _get_k5():
    return pl.kernel(
        _k5_body,
        out_type=jax.ShapeDtypeStruct((T, OUT), jnp.float32),
        mesh=_mesh(),
        compiler_params=pltpu.CompilerParams(needs_layout_passes=False),
        scratch_types=[
            pltpu.VMEM((TPW,), jnp.int32),
            pltpu.VMEM((TPW,), jnp.int32),
            pltpu.VMEM((TPW,), jnp.float32),
            pltpu.VMEM((TPW,), jnp.float32),
            pltpu.VMEM((CH, OUT), jnp.float32),
            pltpu.VMEM((CH, OUT), jnp.float32),
            pltpu.VMEM((CH, OUT), jnp.float32),
            pltpu.VMEM((CH, OUT), jnp.float32),
            pltpu.VMEM((CH, OUT), jnp.float32),
            pltpu.SemaphoreType.DMA,
            pltpu.SemaphoreType.DMA,
        ],
    )


def kernel(x, gate_w, gate_b, expert_w, expert_b):
    B, S, Hd = x.shape
    hs = x.reshape(-1, Hd)
    logits = _router(hs, gate_w, gate_b)
    e0, e1, w0, w1, hist = _get_k2a()(logits.reshape(-1))
    dst0, dst1, blke, blkv, xs = _get_k2b()(hs, e0, e1, hist)
    y = _gmm(blke, blkv, xs, expert_w, expert_b)
    out = _get_k5()(y, dst0, dst1, w0, w1)
    return out.reshape(B, S, OUT), logits


# speedup vs baseline: 3.3218x; 1.0276x over previous
import functools

import jax
import jax.numpy as jnp
from jax import lax
from jax.experimental import pallas as pl
from jax.experimental.pallas import tpu as pltpu, tpu_sc as plsc

H = 1024
OUT = 1024
E = 16
T = 8192
BLK = 512
BLK_SHIFT = 9
NB = 48
P = NB * BLK
NW = 32
TPW = T // NW
CH = 16
NCH = TPW // CH


@functools.lru_cache(maxsize=None)
def _mesh():
    return plsc.VectorSubcoreMesh(
        core_axis_name="c", subcore_axis_name="s", num_cores=2,
        num_subcores=16)


def _wid():
    return lax.axis_index("c") * 16 + lax.axis_index("s")


def _router_body(x_ref, gw_ref, gb_ref, logits_ref):
    logits_ref[...] = lax.dot_general(
        x_ref[...], gw_ref[...], (((1,), (1,)), ((), ())),
        preferred_element_type=jnp.float32) + gb_ref[...]


def _router(hs, gate_w, gate_b):
    gwp = jnp.pad(gate_w, ((0, 128 - E), (0, 0)))
    gbp = jnp.pad(gate_b, (0, 128 - E))
    return pl.pallas_call(
        _router_body,
        grid=(8,),
        in_specs=[
            pl.BlockSpec((T // 8, H), lambda i: (i, 0)),
            pl.BlockSpec((128, H), lambda i: (0, 0)),
            pl.BlockSpec((1, 128), lambda i: (0, 0)),
        ],
        out_specs=pl.BlockSpec((T // 8, 128), lambda i: (i, 0)),
        out_shape=jax.ShapeDtypeStruct((T, 128), jnp.float32),
    )(hs, gwp, gbp.reshape(1, 128))


def _k2a_body(logits_hbm, lg16_hbm, e0_hbm, e1_hbm, w0_hbm, w1_hbm,
              hist_hbm, lg_v, lg16_v, e0_v, e1_v, w0_v, w1_v, hist_v):
    wid = _wid()
    base = wid * TPW
    pltpu.sync_copy(logits_hbm.at[pl.ds(base * 128, TPW * 128)], lg_v)
    lanes = lax.iota(jnp.int32, 16)

    neg = jnp.float32(-jnp.inf)

    def chunk(c, hist):
        a0 = jnp.zeros((16,), jnp.int32)
        a1 = jnp.zeros((16,), jnp.int32)
        w0 = jnp.zeros((16,), jnp.float32)
        w1 = jnp.zeros((16,), jnp.float32)
        for r in range(CH):
            l = lg_v[pl.ds((c * CH + r) * 128, E)]
            lg16_v[pl.ds((c * CH + r) * E, E)] = l
            m0 = jnp.max(l)
            a0s = jnp.min(jnp.where(l == m0, lanes, E))
            lm = jnp.where(lanes == a0s, neg, l)
            m1 = jnp.max(lm)
            a1s = jnp.min(jnp.where(lm == m1, lanes, E))
            t = jnp.exp(jnp.broadcast_to(m1 - m0, (16,)))
            s = 1.0 / (1.0 + t)
            a0 = jnp.where(lanes == r, a0s, a0)
            a1 = jnp.where(lanes == r, a1s, a1)
            w0 = jnp.where(lanes == r, s, w0)
            w1 = jnp.where(lanes == r, t * s, w1)
        e0_v[pl.ds(c * CH, CH)] = a0
        e1_v[pl.ds(c * CH, CH)] = a1
        w0_v[pl.ds(c * CH, CH)] = w0
        w1_v[pl.ds(c * CH, CH)] = w1
        for e in range(E):
            n = jnp.sum((a0 == e).astype(jnp.int32)) + jnp.sum(
                (a1 == e).astype(jnp.int32))
            hist = hist + jnp.where(lanes == e, n, 0)
        return hist

    hist = lax.fori_loop(0, NCH, chunk, jnp.zeros((16,), jnp.int32))
    hist_v[...] = hist
    pltpu.sync_copy(lg16_v, lg16_hbm.at[pl.ds(base * E, TPW * E)])
    pltpu.sync_copy(e0_v, e0_hbm.at[pl.ds(base, TPW)])
    pltpu.sync_copy(e1_v, e1_hbm.at[pl.ds(base, TPW)])
    pltpu.sync_copy(w0_v, w0_hbm.at[pl.ds(base, TPW)])
    pltpu.sync_copy(w1_v, w1_hbm.at[pl.ds(base, TPW)])
    pltpu.sync_copy(hist_v, hist_hbm.at[pl.ds(wid * E, E)])


@functools.lru_cache(maxsize=None)
def _get_k2a():
    return pl.kernel(
        _k2a_body,
        out_type=[
            jax.ShapeDtypeStruct((T * E,), jnp.float32),
            jax.ShapeDtypeStruct((T,), jnp.int32),
            jax.ShapeDtypeStruct((T,), jnp.int32),
            jax.ShapeDtypeStruct((T,), jnp.float32),
            jax.ShapeDtypeStruct((T,), jnp.float32),
            jax.ShapeDtypeStruct((NW * E,), jnp.int32),
        ],
        mesh=_mesh(),
        compiler_params=pltpu.CompilerParams(needs_layout_passes=False),
        scratch_types=[
            pltpu.VMEM((TPW * 128,), jnp.float32),
            pltpu.VMEM((TPW * E,), jnp.float32),
            pltpu.VMEM((TPW,), jnp.int32),
            pltpu.VMEM((TPW,), jnp.int32),
            pltpu.VMEM((TPW,), jnp.float32),
            pltpu.VMEM((TPW,), jnp.float32),
            pltpu.VMEM((E,), jnp.int32),
        ],
    )


def _k2b_body(x_hbm, e0_hbm, e1_hbm, hist_hbm, dst0_hbm, dst1_hbm, blke_hbm,
              blkv_hbm, xs_hbm, hist_v, e0_v, e1_v, d0_v, d1_v, blke_v,
              blkv_v, rows_v, sem, sem2):
    wid = _wid()
    base = wid * TPW
    pltpu.sync_copy(hist_hbm, hist_v)
    pltpu.sync_copy(e0_hbm.at[pl.ds(base, TPW)], e0_v)
    pltpu.sync_copy(e1_hbm.at[pl.ds(base, TPW)], e1_v)
    lanes = lax.iota(jnp.int32, 16)

    tot = jnp.zeros((16,), jnp.int32)
    mybase = jnp.zeros((16,), jnp.int32)
    for wj in range(NW):
        row = hist_v[pl.ds(wj * E, E)]
        tot = tot + row
        mybase = mybase + jnp.where(jnp.int32(wj) < wid, row, 0)
    padded = ((tot + (BLK - 1)) >> BLK_SHIFT) << BLK_SHIFT
    cum = plsc.cumsum(padded)
    start = cum - padded
    basepos = start + mybase

    def chunk(c, cnt):
        k0 = e0_v[pl.ds(c * CH, CH)]
        k1 = e1_v[pl.ds(c * CH, CH)]
        d0 = jnp.zeros((16,), jnp.int32)
        d1 = jnp.zeros((16,), jnp.int32)
        for e in range(E):
            cnt_e = jnp.sum(jnp.where(lanes == e, cnt, 0))
            m0i = (k0 == e).astype(jnp.int32)
            ex0 = plsc.cumsum(m0i) - m0i
            d0 = d0 + (cnt_e + ex0) * m0i
            n0 = jnp.sum(m0i)
            m1i = (k1 == e).astype(jnp.int32)
            ex1 = plsc.cumsum(m1i) - m1i
            d1 = d1 + (cnt_e + n0 + ex1) * m1i
            n1 = jnp.sum(m1i)
            cnt = cnt + jnp.where(lanes == e, n0 + n1, 0)
        d0_v[pl.ds(c * CH, CH)] = d0
        d1_v[pl.ds(c * CH, CH)] = d1
        return cnt

    lax.fori_loop(0, NCH, chunk, basepos)
    pltpu.sync_copy(d0_v, dst0_hbm.at[pl.ds(base, TPW)])
    pltpu.sync_copy(d1_v, dst1_hbm.at[pl.ds(base, TPW)])

    def dchunk(c, _):
        pltpu.sync_copy(x_hbm.at[pl.ds(base + c * CH, CH)], rows_v)
        i0 = d0_v[pl.ds(c * CH, CH)]
        i1 = d1_v[pl.ds(c * CH, CH)]
        cp0 = pltpu.make_async_copy(rows_v, xs_hbm.at[i0], sem)
        cp1 = pltpu.make_async_copy(rows_v, xs_hbm.at[i1], sem2)
        cp0.start()
        cp1.start()
        cp0.wait()
        cp1.wait()
        return 0

    lax.fori_loop(0, NCH, dchunk, 0)

    @pl.when(wid == 0)
    def _blocks():
        total_pad = jnp.sum(padded)
        for g in range(NB // 16):
            pos = (lax.iota(jnp.int32, 16) + g * 16) * BLK
            be = jnp.zeros((16,), jnp.int32)
            for e in range(E):
                end_e = jnp.sum(jnp.where(lanes == e, cum, 0))
                be = be + (pos >= end_e).astype(jnp.int32)
            blke_v[pl.ds(g * 16, 16)] = jnp.minimum(be, E - 1)
            blkv_v[pl.ds(g * 16, 16)] = (pos < total_pad).astype(jnp.int32)
        pltpu.sync_copy(blke_v, blke_hbm)
        pltpu.sync_copy(blkv_v, blkv_hbm)


@functools.lru_cache(maxsize=None)
def _get_k2b():
    return pl.kernel(
        _k2b_body,
        out_type=[
            jax.ShapeDtypeStruct((T,), jnp.int32),
            jax.ShapeDtypeStruct((T,), jnp.int32),
            jax.ShapeDtypeStruct((NB,), jnp.int32),
            jax.ShapeDtypeStruct((NB,), jnp.int32),
            jax.ShapeDtypeStruct((P, H), jnp.float32),
        ],
        mesh=_mesh(),
        compiler_params=pltpu.CompilerParams(needs_layout_passes=False),
        scratch_types=[
            pltpu.VMEM((NW * E,), jnp.int32),
            pltpu.VMEM((TPW,), jnp.int32),
            pltpu.VMEM((TPW,), jnp.int32),
            pltpu.VMEM((TPW,), jnp.int32),
            pltpu.VMEM((TPW,), jnp.int32),
            pltpu.VMEM((NB,), jnp.int32),
            pltpu.VMEM((NB,), jnp.int32),
            pltpu.VMEM((CH, H), jnp.float32),
            pltpu.SemaphoreType.DMA,
            pltpu.SemaphoreType.DMA,
        ],
    )


def _gmm_body(be_ref, bv_ref, xs_ref, ew_ref, eb_ref, y_ref):
    i = pl.program_id(0)

    @pl.when(bv_ref[i] != 0)
    def _():
        y_ref[...] = lax.dot_general(
            xs_ref[...], ew_ref[0], (((1,), (1,)), ((), ())),
            preferred_element_type=jnp.float32) + eb_ref[0]


def _gmm(blke, blkv, xs, expert_w, expert_b):
    grid_spec = pltpu.PrefetchScalarGridSpec(
        num_scalar_prefetch=2,
        grid=(NB,),
        in_specs=[
            pl.BlockSpec((BLK, H), lambda i, be, bv: (i, 0)),
            pl.BlockSpec((1, OUT, H), lambda i, be, bv: (be[i], 0, 0)),
            pl.BlockSpec((1, 1, OUT), lambda i, be, bv: (be[i], 0, 0)),
        ],
        out_specs=pl.BlockSpec((BLK, OUT), lambda i, be, bv: (i, 0)),
    )
    return pl.pallas_call(
        _gmm_body,
        grid_spec=grid_spec,
        out_shape=jax.ShapeDtypeStruct((P, OUT), jnp.float32),
        compiler_params=pltpu.CompilerParams(
            dimension_semantics=("arbitrary",)),
    )(blke, blkv, xs, expert_w, expert_b.reshape(E, 1, OUT))


def _k5_body(y_hbm, dst0_hbm, dst1_hbm, w0_hbm, w1_hbm, out_hbm,
             d0_v, d1_v, w0_v, w1_v, y0a_v, y1a_v, y0b_v, y1b_v, o_v,
             sem_a, sem_b):
    wid = _wid()
    base = wid * TPW
    pltpu.sync_copy(dst0_hbm.at[pl.ds(base, TPW)], d0_v)
    pltpu.sync_copy(dst1_hbm.at[pl.ds(base, TPW)], d1_v)
    pltpu.sync_copy(w0_hbm.at[pl.ds(base, TPW)], w0_v)
    pltpu.sync_copy(w1_hbm.at[pl.ds(base, TPW)], w1_v)
    lanes = lax.iota(jnp.int32, 16)

    UNROLL = 8

    def issue(c, yb0, yb1, sem):
        i0 = d0_v[pl.ds(c * CH, CH)]
        i1 = d1_v[pl.ds(c * CH, CH)]
        pltpu.make_async_copy(y_hbm.at[i0], yb0, sem).start()
        pltpu.make_async_copy(y_hbm.at[i1], yb1, sem).start()

    def wait_pair(c, yb0, yb1, sem):
        i0 = d0_v[pl.ds(c * CH, CH)]
        pltpu.make_async_copy(y_hbm.at[i0], yb0, sem).wait()
        pltpu.make_async_copy(y_hbm.at[i0], yb1, sem).wait()

    def compute(c, yb0, yb1):
        wa = w0_v[pl.ds(c * CH, CH)]
        wb = w1_v[pl.ds(c * CH, CH)]
        for r in range(CH):
            w0r = jnp.sum(jnp.where(lanes == r, wa, 0.0))
            w1r = jnp.sum(jnp.where(lanes == r, wb, 0.0))

            def hloop(h, _, _r=r, _w0=w0r, _w1=w1r):
                for u in range(UNROLL):
                    sl = pl.ds((h * UNROLL + u) * 16, 16)
                    o_v[_r, sl] = yb0[_r, sl] * _w0 + yb1[_r, sl] * _w1
                return 0

            lax.fori_loop(0, OUT // (16 * UNROLL), hloop, 0)
        pltpu.sync_copy(o_v, out_hbm.at[pl.ds(base + c * CH, CH)])

    issue(0, y0a_v, y1a_v, sem_a)

    def chunk(c, _):
        @pl.when(c % 2 == 0)
        def _even():
            wait_pair(c, y0a_v, y1a_v, sem_a)

            @pl.when(c + 1 < NCH)
            def _():
                issue(c + 1, y0b_v, y1b_v, sem_b)

            compute(c, y0a_v, y1a_v)

        @pl.when(c % 2 != 0)
        def _odd():
            wait_pair(c, y0b_v, y1b_v, sem_b)

            @pl.when(c + 1 < NCH)
            def _():
                issue(c + 1, y0a_v, y1a_v, sem_a)

            compute(c, y0b_v, y1b_v)

        return 0

    lax.fori_loop(0, NCH, chunk, 0)


@functools.lru_cache(maxsize=None)
def _get_k5():
    return pl.kernel(
        _k5_body,
        out_type=jax.ShapeDtypeStruct((T, OUT), jnp.float32),
        mesh=_mesh(),
        compiler_params=pltpu.CompilerParams(needs_layout_passes=False),
        scratch_types=[
            pltpu.VMEM((TPW,), jnp.int32),
            pltpu.VMEM((TPW,), jnp.int32),
            pltpu.VMEM((TPW,), jnp.float32),
            pltpu.VMEM((TPW,), jnp.float32),
            pltpu.VMEM((CH, OUT), jnp.float32),
            pltpu.VMEM((CH, OUT), jnp.float32),
            pltpu.VMEM((CH, OUT), jnp.float32),
            pltpu.VMEM((CH, OUT), jnp.float32),
            pltpu.VMEM((CH, OUT), jnp.float32),
            pltpu.SemaphoreType.DMA,
            pltpu.SemaphoreType.DMA,
        ],
    )


def kernel(x, gate_w, gate_b, expert_w, expert_b):
    B, S, Hd = x.shape
    hs = x.reshape(-1, Hd)
    logits128 = _router(hs, gate_w, gate_b)
    lgflat, e0, e1, w0, w1, hist = _get_k2a()(logits128.reshape(-1))
    logits = lgflat.reshape(T, E)
    dst0, dst1, blke, blkv, xs = _get_k2b()(hs, e0, e1, hist)
    y = _gmm(blke, blkv, xs, expert_w, expert_b)
    out = _get_k5()(y, dst0, dst1, w0, w1)
    return out.reshape(B, S, OUT), logits
